# Initial kernel scaffold; baseline (speedup 1.0000x reference)
import jax
import jax.numpy as jnp
from jax.experimental import pallas as pl


def kernel(x, edge_index, edge_attr, u, params):
    raise NotImplementedError("write your pallas kernel here")


# speedup vs baseline: 3.3271x; 3.3271x over previous
import functools

import jax
import jax.numpy as jnp
from jax import lax
from jax.experimental import pallas as pl
from jax.experimental.pallas import tpu as pltpu
from jax.experimental.pallas import tpu_sc as plsc

N_CORES = 2
N_SUBCORES = 16
N_WORKERS = N_CORES * N_SUBCORES
CHUNK = 80
EDGE_BLOCK = 3200


def _relu(v):
    return jnp.maximum(v, 0.0)


def _ln(z, g, b):
    mu = jnp.mean(z, axis=-1, keepdims=True)
    var = jnp.mean((z - mu) ** 2, axis=-1, keepdims=True)
    return (z - mu) / jnp.sqrt(var + 1e-5) * g + b


def _prep_body(x_r, a_r, b_r, u_r, d_r, b1_r, p_r, q_r, cec_r):
    x = x_r[...]
    p_r[...] = jnp.dot(x, a_r[...], preferred_element_type=jnp.float32)
    q_r[...] = jnp.dot(x, b_r[...], preferred_element_type=jnp.float32)
    cec_r[...] = (
        jnp.dot(u_r[...], d_r[...], preferred_element_type=jnp.float32) + b1_r[...]
    )


def _prep_call(x, a, b, u, d, b1):
    n = x.shape[0]
    return pl.pallas_call(
        _prep_body,
        out_shape=[
            jax.ShapeDtypeStruct((n, 64), jnp.float32),
            jax.ShapeDtypeStruct((n, 64), jnp.float32),
            jax.ShapeDtypeStruct((1, 64), jnp.float32),
        ],
    )(x, a, b, u, d, b1)


def _gather_call(p, q, row, col):
    e = row.shape[0]
    epw = e // N_WORKERS
    nchunk = epw // CHUNK
    mesh = plsc.VectorSubcoreMesh(core_axis_name="c", subcore_axis_name="s")

    @functools.partial(
        pl.kernel,
        out_type=[
            jax.ShapeDtypeStruct((e, 64), jnp.float32),
            jax.ShapeDtypeStruct((e, 64), jnp.float32),
        ],
        mesh=mesh,
        scratch_types=[
            pltpu.VMEM((CHUNK,), jnp.int32),
            pltpu.VMEM((CHUNK,), jnp.int32),
            pltpu.VMEM((CHUNK, 64), jnp.float32),
            pltpu.VMEM((CHUNK, 64), jnp.float32),
            pltpu.SemaphoreType.DMA,
            pltpu.SemaphoreType.DMA,
        ],
        compiler_params=pltpu.CompilerParams(use_tc_tiling_on_sc=False),
    )
    def gather_kernel(p_hbm, q_hbm, row_hbm, col_hbm, ga_hbm, gb_hbm,
                      idx_a, idx_b, buf_a, buf_b, sem_a, sem_b):
        wid = lax.axis_index("s") * N_CORES + lax.axis_index("c")
        base = wid * epw

        def body(j, carry):
            off = pl.multiple_of(base + j * CHUNK, CHUNK)
            pltpu.sync_copy(row_hbm.at[pl.ds(off, CHUNK)], idx_a)
            pltpu.sync_copy(col_hbm.at[pl.ds(off, CHUNK)], idx_b)
            ca = pltpu.async_copy(p_hbm.at[idx_a], buf_a, sem_a)
            cb = pltpu.async_copy(q_hbm.at[idx_b], buf_b, sem_b)
            ca.wait()
            cb.wait()
            pltpu.sync_copy(buf_a, ga_hbm.at[pl.ds(off, CHUNK)])
            pltpu.sync_copy(buf_b, gb_hbm.at[pl.ds(off, CHUNK)])
            return carry

        lax.fori_loop(0, nchunk, body, 0)

    return gather_kernel(p, q, row, col)


def _edge_body(ga_r, gb_r, ea_r, c1_r, cec_r, w2_r, b2_r, g_r, bn_r,
               w1d_r, b1d_r, w2d_r, b2d_r, gd_r, bnd_r, eow_r, eob_r,
               le_r, eo_r):
    h = _relu(
        ga_r[...] + gb_r[...]
        + jnp.dot(ea_r[...], c1_r[...], preferred_element_type=jnp.float32)
        + cec_r[...]
    )
    z = _relu(jnp.dot(h, w2_r[...], preferred_element_type=jnp.float32) + b2_r[...])
    le = _ln(z, g_r[...], bn_r[...])
    le_r[...] = le
    d = _relu(jnp.dot(le, w1d_r[...], preferred_element_type=jnp.float32) + b1d_r[...])
    z2 = _relu(jnp.dot(d, w2d_r[...], preferred_element_type=jnp.float32) + b2d_r[...])
    e2 = _ln(z2, gd_r[...], bnd_r[...])
    eo_r[...] = jnp.dot(e2, eow_r[...], preferred_element_type=jnp.float32) + eob_r[...]


def _edge_call(ga, gb, ea, c1, cec, w2, b2, g, bn,
               w1d, b1d, w2d, b2d, gd, bnd, eow, eob):
    e = ga.shape[0]
    grid = (e // EDGE_BLOCK,)

    def blk(r, c):
        return pl.BlockSpec((r, c), lambda i: (i, 0))

    def wblk(r, c):
        return pl.BlockSpec((r, c), lambda i: (0, 0))

    return pl.pallas_call(
        _edge_body,
        grid=grid,
        in_specs=[
            blk(EDGE_BLOCK, 64), blk(EDGE_BLOCK, 64), blk(EDGE_BLOCK, 16),
            wblk(16, 64), wblk(1, 64), wblk(64, 16), wblk(1, 16),
            wblk(1, 16), wblk(1, 16),
            wblk(16, 64), wblk(1, 64), wblk(64, 16), wblk(1, 16),
            wblk(1, 16), wblk(1, 16),
            wblk(16, 16), wblk(1, 16),
        ],
        out_specs=[blk(EDGE_BLOCK, 16), blk(EDGE_BLOCK, 16)],
        out_shape=[
            jax.ShapeDtypeStruct((e, 16), jnp.float32),
            jax.ShapeDtypeStruct((e, 16), jnp.float32),
        ],
    )(ga, gb, ea, c1, cec, w2, b2, g, bn, w1d, b1d, w2d, b2d, gd, bnd, eow, eob)


def _scatter_call(le, col, zeros):
    e = le.shape[0]
    n = zeros.shape[0]
    epw = e // N_WORKERS
    nchunk = epw // CHUNK
    mesh = plsc.VectorSubcoreMesh(core_axis_name="c", subcore_axis_name="s")

    @functools.partial(
        pl.kernel,
        out_type=jax.ShapeDtypeStruct((N_CORES, n, 16), jnp.float32),
        mesh=mesh,
        scratch_types=[
            pltpu.VMEM((CHUNK,), jnp.int32),
            pltpu.VMEM((CHUNK, 16), jnp.float32),
            pltpu.VMEM_SHARED((n, 16), jnp.float32),
        ],
        compiler_params=pltpu.CompilerParams(use_tc_tiling_on_sc=False),
    )
    def scatter_kernel(le_hbm, col_hbm, z_hbm, out_hbm, idx_v, buf, acc):
        cid = lax.axis_index("c")
        sid = lax.axis_index("s")
        wid = sid * N_CORES + cid
        base = wid * epw

        @pl.when(sid == 0)
        def _():
            pltpu.sync_copy(z_hbm, acc)

        plsc.subcore_barrier()

        def body(j, carry):
            off = pl.multiple_of(base + j * CHUNK, CHUNK)
            pltpu.sync_copy(col_hbm.at[pl.ds(off, CHUNK)], idx_v)
            pltpu.sync_copy(le_hbm.at[pl.ds(off, CHUNK)], buf)
            pltpu.sync_copy(buf, acc.at[idx_v], add=True)
            return carry

        lax.fori_loop(0, nchunk, body, 0)
        plsc.subcore_barrier()

        @pl.when(sid == 0)
        def _():
            pltpu.sync_copy(acc, out_hbm.at[cid])

    return scatter_kernel(le, col, zeros)


def _final_body(x_r, a0_r, a1_r, u_r,
                acn_r, wacn_r, ducn_r, b1cn_r, w2cn_r, b2cn_r, gcn_r, bncn_r,
                w1dn_r, b1dn_r, w2dn_r, b2dn_r, gdn_r, bndn_r, vow_r, vob_r,
                wucg_r, wvcg_r, wecg_r, b1cg_r, w2cg_r, b2cg_r, gcg_r, bncg_r,
                w1dg_r, b1dg_r, w2dg_r, b2dg_r, gdg_r, bndg_r, uow_r, uob_r,
                v_out_r, u_out_r, *, n_nodes, n_edges):
    agg = a0_r[...] + a1_r[...]
    u = u_r[...]
    cnc = jnp.dot(u, ducn_r[...], preferred_element_type=jnp.float32) + b1cn_r[...]
    h = _relu(
        jnp.dot(x_r[...], acn_r[...], preferred_element_type=jnp.float32)
        + jnp.dot(agg, wacn_r[...], preferred_element_type=jnp.float32)
        + cnc
    )
    lv = _ln(
        _relu(jnp.dot(h, w2cn_r[...], preferred_element_type=jnp.float32) + b2cn_r[...]),
        gcn_r[...], bncn_r[...],
    )
    d = _relu(jnp.dot(lv, w1dn_r[...], preferred_element_type=jnp.float32) + b1dn_r[...])
    v2 = _ln(
        _relu(jnp.dot(d, w2dn_r[...], preferred_element_type=jnp.float32) + b2dn_r[...]),
        gdn_r[...], bndn_r[...],
    )
    v_out_r[...] = (
        jnp.dot(v2, vow_r[...], preferred_element_type=jnp.float32) + vob_r[...]
    )

    agg_v = jnp.sum(lv, axis=0, keepdims=True) * (1.0 / n_nodes)
    agg_e = jnp.sum(agg, axis=0, keepdims=True) * (1.0 / n_edges)
    hu = _relu(
        jnp.dot(u, wucg_r[...], preferred_element_type=jnp.float32)
        + jnp.dot(agg_v, wvcg_r[...], preferred_element_type=jnp.float32)
        + jnp.dot(agg_e, wecg_r[...], preferred_element_type=jnp.float32)
        + b1cg_r[...]
    )
    lu = _ln(
        _relu(jnp.dot(hu, w2cg_r[...], preferred_element_type=jnp.float32) + b2cg_r[...]),
        gcg_r[...], bncg_r[...],
    )
    du = _relu(jnp.dot(lu, w1dg_r[...], preferred_element_type=jnp.float32) + b1dg_r[...])
    u2 = _ln(
        _relu(jnp.dot(du, w2dg_r[...], preferred_element_type=jnp.float32) + b2dg_r[...]),
        gdg_r[...], bndg_r[...],
    )
    u_out_r[...] = (
        jnp.dot(u2, uow_r[...], preferred_element_type=jnp.float32) + uob_r[...]
    )


def _final_call(x, a0, a1, u, args, n_nodes, n_edges):
    return pl.pallas_call(
        functools.partial(_final_body, n_nodes=n_nodes, n_edges=n_edges),
        out_shape=[
            jax.ShapeDtypeStruct((n_nodes, 32), jnp.float32),
            jax.ShapeDtypeStruct((1, 16), jnp.float32),
        ],
    )(x, a0, a1, u, *args)


def kernel(x, edge_index, edge_attr, u, params):
    n = x.shape[0]
    e = edge_attr.shape[0]
    row = edge_index[0]
    col = edge_index[1]
    ce, cn, cg = params["ce"], params["cn"], params["cg"]
    de, dn, dg = params["de"], params["dn"], params["dg"]

    def r2(v):
        return v.reshape(1, -1)

    p_tab, q_tab, cec = _prep_call(
        x, ce["W1"][0:128], ce["W1"][256:384], u, ce["W1"][544:576], r2(ce["b1"])
    )
    ga, gb = _gather_call(p_tab, q_tab, row, col)
    le, e_out = _edge_call(
        ga, gb, edge_attr,
        ce["W1"][512:528], cec, ce["W2"], r2(ce["b2"]), r2(ce["g"]), r2(ce["bn"]),
        de["W1"], r2(de["b1"]), de["W2"], r2(de["b2"]), r2(de["g"]), r2(de["bn"]),
        params["eo_W"], r2(params["eo_b"]),
    )
    partials = _scatter_call(le, col, jnp.zeros((n, 16), jnp.float32))
    final_args = (
        cn["W1"][0:128], cn["W1"][256:272], cn["W1"][272:304], r2(cn["b1"]),
        cn["W2"], r2(cn["b2"]), r2(cn["g"]), r2(cn["bn"]),
        dn["W1"], r2(dn["b1"]), dn["W2"], r2(dn["b2"]), r2(dn["g"]), r2(dn["bn"]),
        params["vo_W"], r2(params["vo_b"]),
        cg["W1"][0:32], cg["W1"][64:192], cg["W1"][192:208], r2(cg["b1"]),
        cg["W2"], r2(cg["b2"]), r2(cg["g"]), r2(cg["bn"]),
        dg["W1"], r2(dg["b1"]), dg["W2"], r2(dg["b2"]), r2(dg["g"]), r2(dg["bn"]),
        params["uo_W"], r2(params["uo_b"]),
    )
    v_out, u_out = _final_call(x, partials[0], partials[1], u, final_args, n, e)
    return (v_out, e_out, u_out)


# speedup vs baseline: 3.9024x; 1.1729x over previous
import functools

import jax
import jax.numpy as jnp
from jax import lax
from jax.experimental import pallas as pl
from jax.experimental.pallas import tpu as pltpu
from jax.experimental.pallas import tpu_sc as plsc

N_CORES = 2
N_SUBCORES = 16
N_WORKERS = N_CORES * N_SUBCORES
CHUNK = 80
EDGE_BLOCK = 3200


def _relu(v):
    return jnp.maximum(v, 0.0)


def _ln(z, g, b):
    mu = jnp.mean(z, axis=-1, keepdims=True)
    var = jnp.mean((z - mu) ** 2, axis=-1, keepdims=True)
    return (z - mu) / jnp.sqrt(var + 1e-5) * g + b


def _prep_body(x_r, a_r, b_r, u_r, d_r, b1_r, p_r, q_r, cec_r):
    x = x_r[...]
    p_r[...] = jnp.dot(x, a_r[...], preferred_element_type=jnp.float32)
    q_r[...] = jnp.dot(x, b_r[...], preferred_element_type=jnp.float32)
    cec_r[...] = (
        jnp.dot(u_r[...], d_r[...], preferred_element_type=jnp.float32) + b1_r[...]
    )


def _prep_call(x, a, b, u, d, b1):
    n = x.shape[0]
    return pl.pallas_call(
        _prep_body,
        out_shape=[
            jax.ShapeDtypeStruct((n, 64), jnp.float32),
            jax.ShapeDtypeStruct((n, 64), jnp.float32),
            jax.ShapeDtypeStruct((1, 64), jnp.float32),
        ],
    )(x, a, b, u, d, b1)


def _gather_call(p, q, row, col):
    e = row.shape[0]
    epw = e // N_WORKERS
    nchunk = epw // CHUNK
    mesh = plsc.VectorSubcoreMesh(core_axis_name="c", subcore_axis_name="s")

    @functools.partial(
        pl.kernel,
        out_type=[
            jax.ShapeDtypeStruct((e, 64), jnp.float32),
            jax.ShapeDtypeStruct((e, 64), jnp.float32),
        ],
        mesh=mesh,
        scratch_types=[
            pltpu.VMEM((CHUNK,), jnp.int32),
            pltpu.VMEM((CHUNK,), jnp.int32),
            pltpu.VMEM((CHUNK, 64), jnp.float32),
            pltpu.VMEM((CHUNK, 64), jnp.float32),
            pltpu.SemaphoreType.DMA,
            pltpu.SemaphoreType.DMA,
        ],
        compiler_params=pltpu.CompilerParams(use_tc_tiling_on_sc=False),
    )
    def gather_kernel(p_hbm, q_hbm, row_hbm, col_hbm, ga_hbm, gb_hbm,
                      idx_a, idx_b, buf_a, buf_b, sem_a, sem_b):
        wid = lax.axis_index("s") * N_CORES + lax.axis_index("c")
        base = wid * epw

        def body(j, carry):
            off = pl.multiple_of(base + j * CHUNK, CHUNK)
            pltpu.sync_copy(row_hbm.at[pl.ds(off, CHUNK)], idx_a)
            pltpu.sync_copy(col_hbm.at[pl.ds(off, CHUNK)], idx_b)
            ca = pltpu.async_copy(p_hbm.at[idx_a], buf_a, sem_a)
            cb = pltpu.async_copy(q_hbm.at[idx_b], buf_b, sem_b)
            ca.wait()
            cb.wait()
            pltpu.sync_copy(buf_a, ga_hbm.at[pl.ds(off, CHUNK)])
            pltpu.sync_copy(buf_b, gb_hbm.at[pl.ds(off, CHUNK)])
            return carry

        lax.fori_loop(0, nchunk, body, 0)

    return gather_kernel(p, q, row, col)


def _edge_body(ga_r, gb_r, ea_r, c1_r, cec_r, w2_r, b2_r, g_r, bn_r,
               w1d_r, b1d_r, w2d_r, b2d_r, gd_r, bnd_r, eow_r, eob_r,
               le_r, eo_r):
    h = _relu(
        ga_r[...] + gb_r[...]
        + jnp.dot(ea_r[...], c1_r[...], preferred_element_type=jnp.float32)
        + cec_r[...]
    )
    z = _relu(jnp.dot(h, w2_r[...], preferred_element_type=jnp.float32) + b2_r[...])
    le = _ln(z, g_r[...], bn_r[...])
    le_r[...] = le
    d = _relu(jnp.dot(le, w1d_r[...], preferred_element_type=jnp.float32) + b1d_r[...])
    z2 = _relu(jnp.dot(d, w2d_r[...], preferred_element_type=jnp.float32) + b2d_r[...])
    e2 = _ln(z2, gd_r[...], bnd_r[...])
    eo_r[...] = jnp.dot(e2, eow_r[...], preferred_element_type=jnp.float32) + eob_r[...]


def _edge_call(ga, gb, ea, c1, cec, w2, b2, g, bn,
               w1d, b1d, w2d, b2d, gd, bnd, eow, eob):
    e = ga.shape[0]
    grid = (e // EDGE_BLOCK,)

    def blk(r, c):
        return pl.BlockSpec((r, c), lambda i: (i, 0))

    def wblk(r, c):
        return pl.BlockSpec((r, c), lambda i: (0, 0))

    return pl.pallas_call(
        _edge_body,
        grid=grid,
        in_specs=[
            blk(EDGE_BLOCK, 64), blk(EDGE_BLOCK, 64), blk(EDGE_BLOCK, 16),
            wblk(16, 64), wblk(1, 64), wblk(64, 16), wblk(1, 16),
            wblk(1, 16), wblk(1, 16),
            wblk(16, 64), wblk(1, 64), wblk(64, 16), wblk(1, 16),
            wblk(1, 16), wblk(1, 16),
            wblk(16, 16), wblk(1, 16),
        ],
        out_specs=[blk(EDGE_BLOCK, 16), blk(EDGE_BLOCK, 16)],
        out_shape=[
            jax.ShapeDtypeStruct((e, 16), jnp.float32),
            jax.ShapeDtypeStruct((e, 16), jnp.float32),
        ],
    )(ga, gb, ea, c1, cec, w2, b2, g, bn, w1d, b1d, w2d, b2d, gd, bnd, eow, eob)


def _scatter_call(le, col, prev):
    e = le.shape[0]
    n = prev.shape[1]
    epw = e // N_WORKERS
    nchunk = epw // CHUNK
    mesh = plsc.VectorSubcoreMesh(core_axis_name="c", subcore_axis_name="s")

    @functools.partial(
        pl.kernel,
        out_type=jax.ShapeDtypeStruct((N_CORES, n, 16), jnp.float32),
        mesh=mesh,
        scratch_types=[
            pltpu.VMEM((CHUNK,), jnp.int32),
            pltpu.VMEM((CHUNK, 16), jnp.float32),
            pltpu.VMEM_SHARED((n, 16), jnp.float32),
        ],
        compiler_params=pltpu.CompilerParams(use_tc_tiling_on_sc=False),
    )
    def scatter_kernel(le_hbm, col_hbm, z_hbm, out_hbm, idx_v, buf, acc):
        cid = lax.axis_index("c")
        sid = lax.axis_index("s")
        wid = sid * N_CORES + cid
        base = wid * epw

        @pl.when(sid == 0)
        def _():
            pltpu.sync_copy(z_hbm.at[cid], acc)

        plsc.subcore_barrier()

        def body(j, carry):
            off = pl.multiple_of(base + j * CHUNK, CHUNK)
            pltpu.sync_copy(col_hbm.at[pl.ds(off, CHUNK)], idx_v)
            pltpu.sync_copy(le_hbm.at[pl.ds(off, CHUNK)], buf)
            pltpu.sync_copy(buf, acc.at[idx_v], add=True)
            return carry

        lax.fori_loop(0, nchunk, body, 0)
        plsc.subcore_barrier()

        @pl.when(sid == 0)
        def _():
            pltpu.sync_copy(acc, out_hbm.at[cid])

    return scatter_kernel(le, col, prev)


def _final_body(x_r, u_r, *refs, n_nodes, n_edges, n_parts):
    (acn_r, wacn_r, ducn_r, b1cn_r, w2cn_r, b2cn_r, gcn_r, bncn_r,
     w1dn_r, b1dn_r, w2dn_r, b2dn_r, gdn_r, bndn_r, vow_r, vob_r,
     wucg_r, wvcg_r, wecg_r, b1cg_r, w2cg_r, b2cg_r, gcg_r, bncg_r,
     w1dg_r, b1dg_r, w2dg_r, b2dg_r, gdg_r, bndg_r, uow_r, uob_r,
     v_out_r, u_out_r) = refs[n_parts:]
    agg = refs[0][0] + refs[0][1]
    for k in range(1, n_parts):
        agg = agg + refs[k][0] + refs[k][1]
    u = u_r[...]
    cnc = jnp.dot(u, ducn_r[...], preferred_element_type=jnp.float32) + b1cn_r[...]
    h = _relu(
        jnp.dot(x_r[...], acn_r[...], preferred_element_type=jnp.float32)
        + jnp.dot(agg, wacn_r[...], preferred_element_type=jnp.float32)
        + cnc
    )
    lv = _ln(
        _relu(jnp.dot(h, w2cn_r[...], preferred_element_type=jnp.float32) + b2cn_r[...]),
        gcn_r[...], bncn_r[...],
    )
    d = _relu(jnp.dot(lv, w1dn_r[...], preferred_element_type=jnp.float32) + b1dn_r[...])
    v2 = _ln(
        _relu(jnp.dot(d, w2dn_r[...], preferred_element_type=jnp.float32) + b2dn_r[...]),
        gdn_r[...], bndn_r[...],
    )
    v_out_r[...] = (
        jnp.dot(v2, vow_r[...], preferred_element_type=jnp.float32) + vob_r[...]
    )

    agg_v = jnp.sum(lv, axis=0, keepdims=True) * (1.0 / n_nodes)
    agg_e = jnp.sum(agg, axis=0, keepdims=True) * (1.0 / n_edges)
    hu = _relu(
        jnp.dot(u, wucg_r[...], preferred_element_type=jnp.float32)
        + jnp.dot(agg_v, wvcg_r[...], preferred_element_type=jnp.float32)
        + jnp.dot(agg_e, wecg_r[...], preferred_element_type=jnp.float32)
        + b1cg_r[...]
    )
    lu = _ln(
        _relu(jnp.dot(hu, w2cg_r[...], preferred_element_type=jnp.float32) + b2cg_r[...]),
        gcg_r[...], bncg_r[...],
    )
    du = _relu(jnp.dot(lu, w1dg_r[...], preferred_element_type=jnp.float32) + b1dg_r[...])
    u2 = _ln(
        _relu(jnp.dot(du, w2dg_r[...], preferred_element_type=jnp.float32) + b2dg_r[...]),
        gdg_r[...], bndg_r[...],
    )
    u_out_r[...] = (
        jnp.dot(u2, uow_r[...], preferred_element_type=jnp.float32) + uob_r[...]
    )


def _final_call(x, parts, u, args, n_nodes, n_edges):
    return pl.pallas_call(
        functools.partial(
            _final_body, n_nodes=n_nodes, n_edges=n_edges, n_parts=len(parts)
        ),
        out_shape=[
            jax.ShapeDtypeStruct((n_nodes, 32), jnp.float32),
            jax.ShapeDtypeStruct((1, 16), jnp.float32),
        ],
    )(x, u, *parts, *args)


def kernel(x, edge_index, edge_attr, u, params):
    n = x.shape[0]
    e = edge_attr.shape[0]
    row = edge_index[0]
    col = edge_index[1]
    ce, cn, cg = params["ce"], params["cn"], params["cg"]
    de, dn, dg = params["de"], params["dn"], params["dg"]

    def r2(v):
        return v.reshape(1, -1)

    p_tab, q_tab, cec = _prep_call(
        x, ce["W1"][0:128], ce["W1"][256:384], u, ce["W1"][544:576], r2(ce["b1"])
    )
    n_slices = 5
    es = e // n_slices
    partials = jnp.zeros((N_CORES, n, 16), jnp.float32)
    eo_parts = []
    for k in range(n_slices):
        rk = lax.slice(row, (k * es,), ((k + 1) * es,))
        ck = lax.slice(col, (k * es,), ((k + 1) * es,))
        eak = lax.slice(edge_attr, (k * es, 0), ((k + 1) * es, edge_attr.shape[1]))
        ga, gb = _gather_call(p_tab, q_tab, rk, ck)
        le, eo = _edge_call(
            ga, gb, eak,
            ce["W1"][512:528], cec, ce["W2"], r2(ce["b2"]), r2(ce["g"]), r2(ce["bn"]),
            de["W1"], r2(de["b1"]), de["W2"], r2(de["b2"]), r2(de["g"]), r2(de["bn"]),
            params["eo_W"], r2(params["eo_b"]),
        )
        eo_parts.append(eo)
        partials = _scatter_call(le, ck, partials)
    e_out = lax.concatenate(eo_parts, 0)
    final_args = (
        cn["W1"][0:128], cn["W1"][256:272], cn["W1"][272:304], r2(cn["b1"]),
        cn["W2"], r2(cn["b2"]), r2(cn["g"]), r2(cn["bn"]),
        dn["W1"], r2(dn["b1"]), dn["W2"], r2(dn["b2"]), r2(dn["g"]), r2(dn["bn"]),
        params["vo_W"], r2(params["vo_b"]),
        cg["W1"][0:32], cg["W1"][64:192], cg["W1"][192:208], r2(cg["b1"]),
        cg["W2"], r2(cg["b2"]), r2(cg["g"]), r2(cg["bn"]),
        dg["W1"], r2(dg["b1"]), dg["W2"], r2(dg["b2"]), r2(dg["g"]), r2(dg["bn"]),
        params["uo_W"], r2(params["uo_b"]),
    )
    v_out, u_out = _final_call(x, [partials], u, final_args, n, e)
    return (v_out, e_out, u_out)
